# Initial kernel scaffold; baseline (speedup 1.0000x reference)
import jax
import jax.numpy as jnp
from jax.experimental import pallas as pl


def kernel(im_set, s_seq, im_len, s_len):
    raise NotImplementedError("write your pallas kernel here")


# speedup vs baseline: 1.1546x; 1.1546x over previous
import functools

import jax
import jax.numpy as jnp
from jax import lax
from jax.experimental import pallas as pl
from jax.experimental.pallas import tpu as pltpu

_MARGIN = 0.2


def _pool_kernel(im_ref, s_ref, im_len_ref, s_len_ref,
                 pim_ref, ps_ref, diag_ref, *, t_full):
    im = im_ref[...]
    s = s_ref[...]
    tb, r, _ = im.shape
    t = s.shape[1]

    im_len = im_len_ref[...]
    s_len = s_len_ref[...]
    r_idx = lax.broadcasted_iota(jnp.int32, (tb, r, 1), 1)
    t_idx = lax.broadcasted_iota(jnp.int32, (tb, t, 1), 1)
    im_mask = ((r_idx >= 1) & (r_idx < im_len)).astype(jnp.float32)
    s_mask = ((t_idx >= 1) & (t_idx <= t_full - 3)
              & (t_idx < s_len - 2)).astype(jnp.float32)

    pim = jnp.sum(im * im_mask, axis=1)
    ps = jnp.sum(s * s_mask, axis=1)

    diag_ref[...] = jnp.sum(pim * ps, axis=1, keepdims=True)
    pim_ref[...] = pim.astype(jnp.bfloat16)
    ps_ref[...] = ps.astype(jnp.bfloat16)


def _loss_kernel(pim_ref, ps_ref, dcol_ref, drow_ref, out_ref, *,
                 tr, margin):
    i = pl.program_id(0)
    scores = lax.dot_general(
        pim_ref[...], ps_ref[...],
        dimension_numbers=(((1,), (1,)), ((), ())),
        preferred_element_type=jnp.float32)
    trows, b = scores.shape

    row = lax.broadcasted_iota(jnp.int32, (trows, b), 0) + i * tr
    col = lax.broadcasted_iota(jnp.int32, (trows, b), 1)
    off_diag = row != col

    d1 = dcol_ref[...]
    d2 = drow_ref[...]
    cost_s = jnp.where(off_diag, jnp.maximum(margin + scores - d1, 0.0), 0.0)
    cost_im = jnp.where(off_diag, jnp.maximum(margin + scores - d2, 0.0), 0.0)
    partial = jnp.sum(cost_s) + jnp.sum(cost_im)

    @pl.when(i == 0)
    def _init():
        out_ref[...] = jnp.zeros_like(out_ref)

    out_ref[...] += partial.reshape(1, 1)


def kernel(im_set, s_seq, im_len, s_len):
    im_set = jnp.asarray(im_set)
    s_seq = jnp.asarray(s_seq)
    b, r, d = im_set.shape
    b_s, t, _ = s_seq.shape
    assert b == b_s, "contrastive loss requires a square score matrix"
    im_len2 = jnp.asarray(im_len, jnp.int32).reshape(b, 1, 1)
    s_len2 = jnp.asarray(s_len, jnp.int32).reshape(b_s, 1, 1)

    tb = 32
    while b % tb:
        tb //= 2
    block_bytes = tb * (r + t) * d * 4
    vmem_limit = int(min(96 << 20, max(32 << 20, 3 * block_bytes)))

    pim, ps, diag = pl.pallas_call(
        functools.partial(_pool_kernel, t_full=t),
        out_shape=(jax.ShapeDtypeStruct((b, d), jnp.bfloat16),
                   jax.ShapeDtypeStruct((b_s, d), jnp.bfloat16),
                   jax.ShapeDtypeStruct((b, 1), jnp.float32)),
        grid=(b // tb,),
        in_specs=[pl.BlockSpec((tb, r, d), lambda i: (i, 0, 0)),
                  pl.BlockSpec((tb, t, d), lambda i: (i, 0, 0)),
                  pl.BlockSpec((tb, 1, 1), lambda i: (i, 0, 0)),
                  pl.BlockSpec((tb, 1, 1), lambda i: (i, 0, 0))],
        out_specs=(pl.BlockSpec((tb, d), lambda i: (i, 0)),
                   pl.BlockSpec((tb, d), lambda i: (i, 0)),
                   pl.BlockSpec((tb, 1), lambda i: (i, 0))),
        compiler_params=pltpu.CompilerParams(
            dimension_semantics=("parallel",),
            vmem_limit_bytes=vmem_limit),
    )(im_set, s_seq, im_len2, s_len2)

    drow = diag.reshape(1, b)

    tr = 256
    while b % tr:
        tr //= 2
    out = pl.pallas_call(
        functools.partial(_loss_kernel, tr=tr, margin=_MARGIN),
        out_shape=jax.ShapeDtypeStruct((1, 1), jnp.float32),
        grid=(b // tr,),
        in_specs=[pl.BlockSpec((tr, d), lambda i: (i, 0)),
                  pl.BlockSpec((b, d), lambda i: (0, 0)),
                  pl.BlockSpec((tr, 1), lambda i: (i, 0)),
                  pl.BlockSpec((1, b), lambda i: (0, 0))],
        out_specs=pl.BlockSpec((1, 1), lambda i: (0, 0)),
        compiler_params=pltpu.CompilerParams(
            dimension_semantics=("arbitrary",)),
    )(pim, ps, diag, drow)
    return out[0, 0]


# speedup vs baseline: 3.6901x; 3.1959x over previous
import functools

import jax
import jax.numpy as jnp
from jax import lax
from jax.experimental import pallas as pl
from jax.experimental.pallas import tpu as pltpu

_MARGIN = 0.2


def _fused_kernel(im_ref, s_ref, im_len_ref, s_len_ref, out_ref,
                  pim_s, ps_s, *, t_full, tb, nb, margin):
    i = pl.program_id(0)

    @pl.when(i < nb)
    def _pool():
        im = im_ref[...]
        s = s_ref[...]
        r = im.shape[0]
        t = s.shape[0]

        im_len = im_len_ref[...]
        s_len = s_len_ref[...]
        r_idx = lax.broadcasted_iota(jnp.int32, (r, tb, 1), 0)
        t_idx = lax.broadcasted_iota(jnp.int32, (t, tb, 1), 0)
        im_mask = ((r_idx >= 1) & (r_idx < im_len)).astype(jnp.float32)
        s_mask = ((t_idx >= 1) & (t_idx <= t_full - 3)
                  & (t_idx < s_len - 2)).astype(jnp.float32)

        pim = jnp.sum(im * im_mask, axis=0)
        ps = jnp.sum(s * s_mask, axis=0)
        pim_s[pl.ds(i * tb, tb), :] = pim.astype(jnp.bfloat16)
        ps_s[pl.ds(i * tb, tb), :] = ps.astype(jnp.bfloat16)

    @pl.when(i == nb)
    def _loss():
        scores = lax.dot_general(
            pim_s[...], ps_s[...],
            dimension_numbers=(((1,), (1,)), ((), ())),
            preferred_element_type=jnp.float32)
        b = scores.shape[0]
        row = lax.broadcasted_iota(jnp.int32, (b, b), 0)
        col = lax.broadcasted_iota(jnp.int32, (b, b), 1)
        eye = row == col

        diag = jnp.where(eye, scores, 0.0)
        d1 = jnp.sum(diag, axis=1, keepdims=True)
        d2 = jnp.sum(diag, axis=0, keepdims=True)

        cost_s = jnp.where(eye, 0.0, jnp.maximum(margin + scores - d1, 0.0))
        cost_im = jnp.where(eye, 0.0, jnp.maximum(margin + scores - d2, 0.0))
        total = jnp.sum(cost_s) + jnp.sum(cost_im)
        out_ref[...] = total.reshape(1, 1)


def kernel(im_set, s_seq, im_len, s_len):
    im_set = jnp.asarray(im_set)
    s_seq = jnp.asarray(s_seq)
    b, r, d = im_set.shape
    b_s, t, _ = s_seq.shape
    assert b == b_s, "contrastive loss requires a square score matrix"
    im_len2 = jnp.asarray(im_len, jnp.int32).reshape(1, b, 1)
    s_len2 = jnp.asarray(s_len, jnp.int32).reshape(1, b_s, 1)
    im_t = jnp.transpose(im_set, (1, 0, 2))
    s_t = jnp.transpose(s_seq, (1, 0, 2))

    tb = 32
    while b % tb:
        tb //= 2
    nb = b // tb
    clamp = nb - 1

    out = pl.pallas_call(
        functools.partial(_fused_kernel, t_full=t, tb=tb, nb=nb,
                          margin=_MARGIN),
        out_shape=jax.ShapeDtypeStruct((1, 1), jnp.float32),
        grid=(nb + 1,),
        in_specs=[
            pl.BlockSpec((r, tb, d), lambda i: (0, jnp.minimum(i, clamp), 0)),
            pl.BlockSpec((t, tb, d), lambda i: (0, jnp.minimum(i, clamp), 0)),
            pl.BlockSpec((1, tb, 1), lambda i: (0, jnp.minimum(i, clamp), 0)),
            pl.BlockSpec((1, tb, 1), lambda i: (0, jnp.minimum(i, clamp), 0)),
        ],
        out_specs=pl.BlockSpec((1, 1), lambda i: (0, 0)),
        scratch_shapes=[pltpu.VMEM((b, d), jnp.bfloat16),
                        pltpu.VMEM((b, d), jnp.bfloat16)],
        compiler_params=pltpu.CompilerParams(
            dimension_semantics=("arbitrary",),
            vmem_limit_bytes=60 << 20),
    )(im_t, s_t, im_len2, s_len2)
    return out[0, 0]
